# baseline (speedup 1.0000x reference)
import functools

import jax
import jax.numpy as jnp
from jax import lax
from jax.experimental import pallas as pl
from jax.experimental.pallas import tpu as pltpu
from jax.experimental.pallas import tpu_sc as plsc

_NULL_CLASS = 1000
_LANES = 16
_NW = 32
_NS = 16
_CHUNK = 128
_N_HBM = 0
_EXT_V = 1536


def kernel(labels, train, force_drop_ids, table):
    del train
    (B,) = labels.shape
    V, D = table.shape
    BPW = B // _NW
    NCH = BPW // _CHUNK
    RPT = _EXT_V // _NS

    null_rep = jnp.broadcast_to(table[_NULL_CLASS], (_EXT_V - V, D))
    table_ext = jnp.concatenate([table, null_rep], axis=0)

    mesh = plsc.VectorSubcoreMesh(core_axis_name="c", subcore_axis_name="s")

    @functools.partial(
        pl.kernel,
        mesh=mesh,
        out_type=jax.ShapeDtypeStruct((B, D), jnp.float32),
        scratch_types=[
            pltpu.VMEM((BPW,), jnp.int32),
            pltpu.VMEM((BPW,), jnp.int32),
            pltpu.VMEM((BPW,), jnp.int32),
            pltpu.VMEM((BPW, D), jnp.float32),
            pltpu.VMEM_SHARED((_EXT_V, D), jnp.float32),
            pltpu.SemaphoreType.DMA,
            pltpu.SemaphoreType.DMA,
            pltpu.SemaphoreType.DMA,
        ],
    )
    def emb(labels_hbm, drop_hbm, table_hbm, out_hbm,
            lab_v, drp_v, idx_v, rows_v, shared_v, gsem, hsem, wsem):
        sid = lax.axis_index("s")
        wid = sid * 2 + lax.axis_index("c")
        base = wid * BPW
        stage = pltpu.async_copy(
            table_hbm.at[pl.ds(sid * RPT, RPT)],
            shared_v.at[pl.ds(sid * RPT, RPT)],
            wsem,
        )
        in0 = pltpu.async_copy(labels_hbm.at[pl.ds(base, BPW)], lab_v, gsem)
        in1 = pltpu.async_copy(drop_hbm.at[pl.ds(base, BPW)], drp_v, gsem)
        in0.wait()
        in1.wait()
        lane = lax.iota(jnp.int32, _LANES)

        def idx_body(i, _):
            off = i * _LANES
            lab = lab_v[pl.ds(off, _LANES)]
            drp = drp_v[pl.ds(off, _LANES)]
            null_row = lane + (_NULL_CLASS + off)
            idx_v[pl.ds(off, _LANES)] = jnp.where(drp == 1, null_row, lab)
            return 0

        vec_per_chunk = _CHUNK // _LANES
        lax.fori_loop(0, vec_per_chunk, idx_body, 0)
        g0 = pltpu.async_copy(
            table_hbm.at[idx_v.at[pl.ds(0, _CHUNK)]],
            rows_v.at[pl.ds(0, _CHUNK)],
            hsem,
        )
        lax.fori_loop(vec_per_chunk, BPW // _LANES, idx_body, 0)
        stage.wait()
        plsc.subcore_barrier()
        gathers = [g0]
        for j in range(1, NCH):
            gathers.append(
                pltpu.async_copy(
                    shared_v.at[idx_v.at[pl.ds(j * _CHUNK, _CHUNK)]],
                    rows_v.at[pl.ds(j * _CHUNK, _CHUNK)],
                    gsem,
                )
            )
        writes = []
        for j in range(NCH):
            gathers[j].wait()
            writes.append(
                pltpu.async_copy(
                    rows_v.at[pl.ds(j * _CHUNK, _CHUNK)],
                    out_hbm.at[pl.ds(base + j * _CHUNK, _CHUNK)],
                    wsem,
                )
            )
        for w in writes:
            w.wait()

    return emb(labels, force_drop_ids, table_ext)


# speedup vs baseline: 1.1128x; 1.1128x over previous
import functools

import jax
import jax.numpy as jnp
from jax import lax
from jax.experimental import pallas as pl
from jax.experimental.pallas import tpu as pltpu
from jax.experimental.pallas import tpu_sc as plsc

_NULL_CLASS = 1000
_LANES = 16
_NW = 32
_NS = 16
_CHUNK = 128
_N_HBM = 0
_EXT_V = 1536


def kernel(labels, train, force_drop_ids, table):
    del train
    (B,) = labels.shape
    V, D = table.shape
    BPW = B // _NW
    NCH = BPW // _CHUNK
    RPT = _EXT_V // _NS

    null_rep = jnp.broadcast_to(table[_NULL_CLASS], (_EXT_V - V, D))
    table_ext = jnp.concatenate([table, null_rep], axis=0)

    mesh = plsc.VectorSubcoreMesh(core_axis_name="c", subcore_axis_name="s")

    @functools.partial(
        pl.kernel,
        mesh=mesh,
        out_type=jax.ShapeDtypeStruct((B, D), jnp.float32),
        scratch_types=[
            pltpu.VMEM((BPW,), jnp.int32),
            pltpu.VMEM((BPW,), jnp.int32),
            pltpu.VMEM((BPW,), jnp.int32),
            pltpu.VMEM((BPW, D), jnp.float32),
            pltpu.VMEM_SHARED((_EXT_V, D), jnp.float32),
            pltpu.SemaphoreType.DMA,
            pltpu.SemaphoreType.DMA,
            pltpu.SemaphoreType.DMA,
        ],
    )
    def emb(labels_hbm, drop_hbm, table_hbm, out_hbm,
            lab_v, drp_v, idx_v, rows_v, shared_v, gsem, hsem, wsem):
        sid = lax.axis_index("s")
        wid = sid * 2 + lax.axis_index("c")
        base = wid * BPW
        stage = pltpu.async_copy(
            table_hbm.at[pl.ds(sid * RPT, RPT)],
            shared_v.at[pl.ds(sid * RPT, RPT)],
            wsem,
        )
        in0 = pltpu.async_copy(labels_hbm.at[pl.ds(base, BPW)], lab_v, gsem)
        in1 = pltpu.async_copy(drop_hbm.at[pl.ds(base, BPW)], drp_v, gsem)
        in0.wait()
        in1.wait()
        lane = lax.iota(jnp.int32, _LANES)

        def idx_body(i, _):
            off = i * _LANES
            lab = lab_v[pl.ds(off, _LANES)]
            drp = drp_v[pl.ds(off, _LANES)]
            null_row = lane + (_NULL_CLASS + off)
            idx_v[pl.ds(off, _LANES)] = jnp.where(drp == 1, null_row, lab)
            return 0

        lax.fori_loop(0, BPW // _LANES, idx_body, 0)
        stage.wait()
        plsc.subcore_barrier()
        chunk_offs = [0, 128, 256, 384, 480]
        chunk_lens = [128, 128, 128, 96, 32]
        gathers = [
            pltpu.async_copy(
                shared_v.at[idx_v.at[pl.ds(o, n)]],
                rows_v.at[pl.ds(o, n)],
                gsem,
            )
            for o, n in zip(chunk_offs, chunk_lens)
        ]
        writes = []
        for g, o, n in zip(gathers, chunk_offs, chunk_lens):
            g.wait()
            writes.append(
                pltpu.async_copy(
                    rows_v.at[pl.ds(o, n)],
                    out_hbm.at[pl.ds(base + o, n)],
                    wsem,
                )
            )
        for w in writes:
            w.wait()

    return emb(labels, force_drop_ids, table_ext)


# speedup vs baseline: 1.1157x; 1.0025x over previous
import functools

import jax
import jax.numpy as jnp
from jax import lax
from jax.experimental import pallas as pl
from jax.experimental.pallas import tpu as pltpu
from jax.experimental.pallas import tpu_sc as plsc

_NULL_CLASS = 1000
_LANES = 16
_NW = 32
_NS = 16
_CHUNK = 128
_N_HBM = 0
_EXT_V = 1536


def kernel(labels, train, force_drop_ids, table):
    del train
    (B,) = labels.shape
    V, D = table.shape
    BPW = B // _NW
    NCH = BPW // _CHUNK
    RPT = _EXT_V // _NS

    null_rep = jnp.broadcast_to(table[_NULL_CLASS], (_EXT_V - V, D))
    table_ext = jnp.concatenate([table, null_rep], axis=0)

    mesh = plsc.VectorSubcoreMesh(core_axis_name="c", subcore_axis_name="s")

    @functools.partial(
        pl.kernel,
        mesh=mesh,
        out_type=jax.ShapeDtypeStruct((B, D), jnp.float32),
        scratch_types=[
            pltpu.VMEM((BPW,), jnp.int32),
            pltpu.VMEM((BPW,), jnp.int32),
            pltpu.VMEM((BPW,), jnp.int32),
            pltpu.VMEM((BPW, D), jnp.float32),
            pltpu.VMEM_SHARED((_EXT_V, D), jnp.float32),
            pltpu.SemaphoreType.DMA,
            pltpu.SemaphoreType.DMA,
            pltpu.SemaphoreType.DMA,
        ],
    )
    def emb(labels_hbm, drop_hbm, table_hbm, out_hbm,
            lab_v, drp_v, idx_v, rows_v, shared_v, gsem, hsem, wsem):
        sid = lax.axis_index("s")
        wid = sid * 2 + lax.axis_index("c")
        base = wid * BPW
        stage = pltpu.async_copy(
            table_hbm.at[pl.ds(sid * RPT, RPT)],
            shared_v.at[pl.ds(sid * RPT, RPT)],
            wsem,
        )
        in0 = pltpu.async_copy(labels_hbm.at[pl.ds(base, BPW)], lab_v, gsem)
        in1 = pltpu.async_copy(drop_hbm.at[pl.ds(base, BPW)], drp_v, gsem)
        in0.wait()
        in1.wait()
        lane = lax.iota(jnp.int32, _LANES)

        @plsc.parallel_loop(0, BPW // _LANES, step=1, unroll=4)
        def _idx_body(i):
            off = i * _LANES
            lab = lab_v[pl.ds(off, _LANES)]
            drp = drp_v[pl.ds(off, _LANES)]
            null_row = lane + (_NULL_CLASS + off)
            idx_v[pl.ds(off, _LANES)] = jnp.where(drp == 1, null_row, lab)
        stage.wait()
        plsc.subcore_barrier()
        chunk_offs = [0, 128, 256, 384, 480]
        chunk_lens = [128, 128, 128, 96, 32]
        gathers = [
            pltpu.async_copy(
                shared_v.at[idx_v.at[pl.ds(o, n)]],
                rows_v.at[pl.ds(o, n)],
                gsem,
            )
            for o, n in zip(chunk_offs, chunk_lens)
        ]
        writes = []
        for g, o, n in zip(gathers, chunk_offs, chunk_lens):
            g.wait()
            writes.append(
                pltpu.async_copy(
                    rows_v.at[pl.ds(o, n)],
                    out_hbm.at[pl.ds(base + o, n)],
                    wsem,
                )
            )
        for w in writes:
            w.wait()

    return emb(labels, force_drop_ids, table_ext)
